# Initial kernel scaffold; baseline (speedup 1.0000x reference)
import jax
import jax.numpy as jnp
from jax.experimental import pallas as pl


def kernel(x_train, edge_index, x_ori, mask, W1, b1, W2, b2, Wq, bq, Wk, bk, Wv, bv, Wd, bd):
    raise NotImplementedError("write your pallas kernel here")


# speedup vs baseline: 4.6076x; 4.6076x over previous
import functools

import jax
import jax.numpy as jnp
from jax import lax
from jax.experimental import pallas as pl
from jax.experimental.pallas import tpu as pltpu
from jax.experimental.pallas import tpu_sc as plsc

N = 10000
E = 160000
IN = 128
H = 64
HP = 128
QK = 32

NC, NS = 2, 16
NW = NC * NS
EPT = E // NW
BLK = 100
NB = EPT // BLK
NPAD = 10240
RPT = NPAD // NS
ZR = 64

BR = 1000
GRID = N // BR
BRA = 400


def _sc_mesh():
    return plsc.VectorSubcoreMesh(
        core_axis_name="c", subcore_axis_name="s", num_cores=NC, num_subcores=NS
    )


@functools.partial(
    pl.kernel,
    out_type=jax.ShapeDtypeStruct((NC, NPAD, HP), jnp.float32),
    mesh=_sc_mesh(),
    scratch_types=[
        pltpu.VMEM((NB, BLK), jnp.int32),
        pltpu.VMEM((BLK, HP), jnp.float32),
        pltpu.VMEM_SHARED((NPAD, HP), jnp.float32),
        pltpu.VMEM((ZR, HP), jnp.float32),
    ],
)
def _deg_kernel(rows_hbm, out_hbm, ridx, ones_v, acc, zbuf):
    c = lax.axis_index("c")
    s = lax.axis_index("s")
    wid = s * NC + c

    def zf(i, _):
        for k in range(HP // 16):
            zbuf[i, pl.ds(k * 16, 16)] = jnp.zeros((16,), jnp.float32)
        return 0

    lax.fori_loop(0, ZR, zf, 0)

    def of(i, _):
        for k in range(HP // 16):
            ones_v[i, pl.ds(k * 16, 16)] = jnp.full((16,), 1.0, jnp.float32)
        return 0

    lax.fori_loop(0, BLK, of, 0)

    def zc(i, _):
        pltpu.sync_copy(zbuf, acc.at[pl.ds(s * RPT + i * ZR, ZR), :])
        return 0

    lax.fori_loop(0, RPT // ZR, zc, 0)
    pltpu.sync_copy(rows_hbm.at[wid], ridx)
    plsc.subcore_barrier()

    def body(j, _):
        pltpu.sync_copy(ones_v, acc.at[ridx.at[j]], add=True)
        return 0

    lax.fori_loop(0, NB, body, 0)
    plsc.subcore_barrier()
    pltpu.sync_copy(acc.at[pl.ds(s * RPT, RPT), :],
                    out_hbm.at[c, pl.ds(s * RPT, RPT), :])


@functools.partial(
    pl.kernel,
    out_type=jax.ShapeDtypeStruct((NC, NPAD, HP), jnp.float32),
    mesh=_sc_mesh(),
    scratch_types=[
        pltpu.VMEM((NB, BLK), jnp.int32),
        pltpu.VMEM((NB, BLK), jnp.int32),
        pltpu.VMEM((BLK, HP), jnp.float32),
        pltpu.VMEM((BLK, HP), jnp.float32),
        pltpu.VMEM_SHARED((NPAD, HP), jnp.float32),
        pltpu.VMEM((ZR, HP), jnp.float32),
        pltpu.SemaphoreType.DMA,
        pltpu.SemaphoreType.DMA,
    ],
)
def _prop_kernel(xs_hbm, rows_hbm, cols_hbm, out_hbm,
                 ridx, cidx, buf0, buf1, acc, zbuf, sem0, sem1):
    c = lax.axis_index("c")
    s = lax.axis_index("s")
    wid = s * NC + c

    def zf(i, _):
        for k in range(HP // 16):
            zbuf[i, pl.ds(k * 16, 16)] = jnp.zeros((16,), jnp.float32)
        return 0

    lax.fori_loop(0, ZR, zf, 0)

    def zc(i, _):
        pltpu.sync_copy(zbuf, acc.at[pl.ds(s * RPT + i * ZR, ZR), :])
        return 0

    lax.fori_loop(0, RPT // ZR, zc, 0)
    pltpu.sync_copy(rows_hbm.at[wid], ridx)
    pltpu.sync_copy(cols_hbm.at[wid], cidx)
    plsc.subcore_barrier()

    pltpu.async_copy(xs_hbm.at[ridx.at[0]], buf0, sem0)
    pltpu.async_copy(xs_hbm.at[ridx.at[1]], buf1, sem1)

    def body(j2, _):
        j = 2 * j2
        pltpu.make_async_copy(xs_hbm.at[ridx.at[j]], buf0, sem0).wait()
        pltpu.sync_copy(buf0, acc.at[cidx.at[j]], add=True)
        pltpu.async_copy(xs_hbm.at[ridx.at[j + 2]], buf0, sem0)
        pltpu.make_async_copy(xs_hbm.at[ridx.at[j + 1]], buf1, sem1).wait()
        pltpu.sync_copy(buf1, acc.at[cidx.at[j + 1]], add=True)
        pltpu.async_copy(xs_hbm.at[ridx.at[j + 3]], buf1, sem1)
        return 0

    lax.fori_loop(0, NB // 2 - 1, body, 0)
    j = NB - 2
    pltpu.make_async_copy(xs_hbm.at[ridx.at[j]], buf0, sem0).wait()
    pltpu.sync_copy(buf0, acc.at[cidx.at[j]], add=True)
    pltpu.make_async_copy(xs_hbm.at[ridx.at[j + 1]], buf1, sem1).wait()
    pltpu.sync_copy(buf1, acc.at[cidx.at[j + 1]], add=True)
    plsc.subcore_barrier()
    pltpu.sync_copy(acc.at[pl.ds(s * RPT, RPT), :],
                    out_hbm.at[c, pl.ds(s * RPT, RPT), :])


def _mlp_body(xt_ref, xo_ref, deg_ref, w1, b1, w2, b2,
              ht_ref, xs1_ref, dinv_ref, qsum_ref):
    i = pl.program_id(0)

    def mlp(x):
        h = jnp.maximum(
            jnp.dot(x, w1[...], preferred_element_type=jnp.float32) + b1[...], 0.0)
        return jnp.maximum(
            jnp.dot(h, w2[...], preferred_element_type=jnp.float32) + b2[...], 0.0)

    ht = mlp(xt_ref[...])
    ho = mlp(xo_ref[...])
    deg = deg_ref[0][:, 0:1] + deg_ref[1][:, 0:1]
    dinv = lax.rsqrt(jnp.maximum(deg, 1.0))
    ht_ref[...] = ht
    xs1_ref[...] = jnp.concatenate(
        [ht * dinv, jnp.zeros((BR, HP - H), jnp.float32)], axis=1)
    dinv_ref[...] = dinv

    @pl.when(i == 0)
    def _():
        qsum_ref[...] = jnp.zeros((8, H), jnp.float32)

    qsum_ref[0:1, :] = qsum_ref[0:1, :] + jnp.sum(ho, axis=0, keepdims=True)


_mlp_call = pl.pallas_call(
    _mlp_body,
    grid=(GRID,),
    in_specs=[
        pl.BlockSpec((BR, IN), lambda i: (i, 0)),
        pl.BlockSpec((BR, IN), lambda i: (i, 0)),
        pl.BlockSpec((NC, BR, HP), lambda i: (0, i, 0)),
        pl.BlockSpec((IN, H), lambda i: (0, 0)),
        pl.BlockSpec((1, H), lambda i: (0, 0)),
        pl.BlockSpec((H, H), lambda i: (0, 0)),
        pl.BlockSpec((1, H), lambda i: (0, 0)),
    ],
    out_specs=[
        pl.BlockSpec((BR, H), lambda i: (i, 0)),
        pl.BlockSpec((BR, HP), lambda i: (i, 0)),
        pl.BlockSpec((BR, 1), lambda i: (i, 0)),
        pl.BlockSpec((8, H), lambda i: (0, 0)),
    ],
    out_shape=[
        jax.ShapeDtypeStruct((N, H), jnp.float32),
        jax.ShapeDtypeStruct((N, HP), jnp.float32),
        jax.ShapeDtypeStruct((N, 1), jnp.float32),
        jax.ShapeDtypeStruct((8, H), jnp.float32),
    ],
)


def _combine_body(p_ref, ht_ref, dinv_ref, af_ref, x1_ref, xs2_ref):
    dinv = dinv_ref[...]
    out = (p_ref[0][:, :H] + p_ref[1][:, :H]) * dinv
    ht = ht_ref[...]
    af_ref[...] = ht + out
    x1 = ht - out
    x1_ref[...] = x1
    xs2_ref[...] = jnp.concatenate(
        [x1 * dinv, jnp.zeros((BR, HP - H), jnp.float32)], axis=1)


_combine_call = pl.pallas_call(
    _combine_body,
    grid=(GRID,),
    in_specs=[
        pl.BlockSpec((NC, BR, HP), lambda i: (0, i, 0)),
        pl.BlockSpec((BR, H), lambda i: (i, 0)),
        pl.BlockSpec((BR, 1), lambda i: (i, 0)),
    ],
    out_specs=[
        pl.BlockSpec((BR, H), lambda i: (i, 0)),
        pl.BlockSpec((BR, H), lambda i: (i, 0)),
        pl.BlockSpec((BR, HP), lambda i: (i, 0)),
    ],
    out_shape=[
        jax.ShapeDtypeStruct((N, H), jnp.float32),
        jax.ShapeDtypeStruct((N, H), jnp.float32),
        jax.ShapeDtypeStruct((N, HP), jnp.float32),
    ],
)


def _attn_body(af_ref, x1_ref, p2_ref, dinv_ref, qsum_ref,
               wq, bq, wk, bk, wv, bv, wd, bd, z_ref, xhat_ref):
    dinv = dinv_ref[...]
    x2 = x1_ref[...] - (p2_ref[0][:, :H] + p2_ref[1][:, :H]) * dinv
    qg = jnp.dot(qsum_ref[0:1, :] * (1.0 / N), wq[...],
                 preferred_element_type=jnp.float32) + bq[...]
    scale = 1.0 / (H ** 0.5)

    def kv(l):
        kj = jnp.dot(l, wk[...], preferred_element_type=jnp.float32) + bk[...]
        sj = lax.dot_general(kj, qg, (((1,), (1,)), ((), ())),
                             preferred_element_type=jnp.float32)
        vj = jnp.dot(l, wv[...], preferred_element_type=jnp.float32) + bv[...]
        return scale * sj, vj

    s0, v0 = kv(af_ref[...])
    s1, v1 = kv(x1_ref[...])
    s2, v2 = kv(x2)
    m = jnp.maximum(jnp.maximum(s0, s1), s2)
    e0 = jnp.exp(s0 - m)
    e1 = jnp.exp(s1 - m)
    e2 = jnp.exp(s2 - m)
    z = (e0 * v0 + e1 * v1 + e2 * v2) / (e0 + e1 + e2)
    z_ref[...] = z
    xhat_ref[...] = jnp.dot(z, wd[...], preferred_element_type=jnp.float32) + bd[...]


_attn_call = pl.pallas_call(
    _attn_body,
    grid=(GRID,),
    in_specs=[
        pl.BlockSpec((BR, H), lambda i: (i, 0)),
        pl.BlockSpec((BR, H), lambda i: (i, 0)),
        pl.BlockSpec((NC, BR, HP), lambda i: (0, i, 0)),
        pl.BlockSpec((BR, 1), lambda i: (i, 0)),
        pl.BlockSpec((8, H), lambda i: (0, 0)),
        pl.BlockSpec((H, QK), lambda i: (0, 0)),
        pl.BlockSpec((1, QK), lambda i: (0, 0)),
        pl.BlockSpec((H, QK), lambda i: (0, 0)),
        pl.BlockSpec((1, QK), lambda i: (0, 0)),
        pl.BlockSpec((H, H), lambda i: (0, 0)),
        pl.BlockSpec((1, H), lambda i: (0, 0)),
        pl.BlockSpec((H, IN), lambda i: (0, 0)),
        pl.BlockSpec((1, IN), lambda i: (0, 0)),
    ],
    out_specs=[
        pl.BlockSpec((BR, H), lambda i: (i, 0)),
        pl.BlockSpec((BR, IN), lambda i: (i, 0)),
    ],
    out_shape=[
        jax.ShapeDtypeStruct((N, H), jnp.float32),
        jax.ShapeDtypeStruct((N, IN), jnp.float32),
    ],
)


def _adj_body(zi_ref, zfull_ref, out_ref):
    out_ref[...] = lax.dot_general(
        zi_ref[...], zfull_ref[...], (((1,), (1,)), ((), ())),
        preferred_element_type=jnp.float32)


_adj_call = pl.pallas_call(
    _adj_body,
    grid=(N // BRA,),
    in_specs=[
        pl.BlockSpec((BRA, H), lambda i: (i, 0)),
        pl.BlockSpec((N, H), lambda i: (0, 0)),
    ],
    out_specs=pl.BlockSpec((BRA, N), lambda i: (i, 0)),
    out_shape=jax.ShapeDtypeStruct((N, N), jnp.float32),
)


def kernel(x_train, edge_index, x_ori, mask,
           W1, b1, W2, b2, Wq, bq, Wk, bk, Wv, bv, Wd, bd):
    er = edge_index.reshape(2, NW, NB, BLK)
    rows = er[0]
    cols = er[1]

    degp = _deg_kernel(rows)

    h_train, xs1, dinv, qsum = _mlp_call(
        x_train, x_ori, degp, W1, b1.reshape(1, H), W2, b2.reshape(1, H))

    p1 = _prop_kernel(xs1, rows, cols)
    a_feat, x1, xs2 = _combine_call(p1, h_train, dinv)
    p2 = _prop_kernel(xs2, rows, cols)

    z, x_hat = _attn_call(
        a_feat, x1, p2, dinv, qsum,
        Wq, bq.reshape(1, QK), Wk, bk.reshape(1, QK),
        Wv, bv.reshape(1, H), Wd, bd.reshape(1, IN))

    adj_hat = _adj_call(z, z)
    return (x_hat, adj_hat)


# speedup vs baseline: 5.1001x; 1.1069x over previous
import functools

import jax
import jax.numpy as jnp
from jax import lax
from jax.experimental import pallas as pl
from jax.experimental.pallas import tpu as pltpu
from jax.experimental.pallas import tpu_sc as plsc

N = 10000
E = 160000
IN = 128
H = 64
HP = 64
DW = 16
QK = 32

NC, NS = 2, 16
NW = NC * NS
EPT = E // NW
BLK = 100
NB = EPT // BLK
NPAD = 10240
RPT = NPAD // NS
ZR = 64

BR = 1000
GRID = N // BR
BRA = 400


def _sc_mesh():
    return plsc.VectorSubcoreMesh(
        core_axis_name="c", subcore_axis_name="s", num_cores=NC, num_subcores=NS
    )


@functools.partial(
    pl.kernel,
    out_type=jax.ShapeDtypeStruct((NC, NPAD, DW), jnp.float32),
    mesh=_sc_mesh(),
    scratch_types=[
        pltpu.VMEM((NB, BLK), jnp.int32),
        pltpu.VMEM((BLK, DW), jnp.float32),
        pltpu.VMEM_SHARED((NPAD, DW), jnp.float32),
        pltpu.VMEM((ZR, DW), jnp.float32),
    ],
    compiler_params=pltpu.CompilerParams(use_tc_tiling_on_sc=False),
)
def _deg_kernel(rows_hbm, out_hbm, ridx, ones_v, acc, zbuf):
    c = lax.axis_index("c")
    s = lax.axis_index("s")
    wid = s * NC + c

    def zf(i, _):
        for k in range(DW // 16):
            zbuf[i, pl.ds(k * 16, 16)] = jnp.zeros((16,), jnp.float32)
        return 0

    lax.fori_loop(0, ZR, zf, 0)

    def of(i, _):
        for k in range(DW // 16):
            ones_v[i, pl.ds(k * 16, 16)] = jnp.full((16,), 1.0, jnp.float32)
        return 0

    lax.fori_loop(0, BLK, of, 0)

    def zc(i, _):
        pltpu.sync_copy(zbuf, acc.at[pl.ds(s * RPT + i * ZR, ZR), :])
        return 0

    lax.fori_loop(0, RPT // ZR, zc, 0)
    pltpu.sync_copy(rows_hbm.at[wid], ridx)
    plsc.subcore_barrier()

    def body(j, _):
        pltpu.sync_copy(ones_v, acc.at[ridx.at[j]], add=True)
        return 0

    lax.fori_loop(0, NB, body, 0)
    plsc.subcore_barrier()
    pltpu.sync_copy(acc.at[pl.ds(s * RPT, RPT), :],
                    out_hbm.at[c, pl.ds(s * RPT, RPT), :])


@functools.partial(
    pl.kernel,
    out_type=jax.ShapeDtypeStruct((NC, NPAD, HP), jnp.float32),
    mesh=_sc_mesh(),
    scratch_types=[
        pltpu.VMEM((NB, BLK), jnp.int32),
        pltpu.VMEM((NB, BLK), jnp.int32),
        pltpu.VMEM((BLK, HP), jnp.float32),
        pltpu.VMEM((BLK, HP), jnp.float32),
        pltpu.VMEM_SHARED((NPAD, HP), jnp.float32),
        pltpu.VMEM((ZR, HP), jnp.float32),
        pltpu.SemaphoreType.DMA,
        pltpu.SemaphoreType.DMA,
    ],
    compiler_params=pltpu.CompilerParams(use_tc_tiling_on_sc=False),
)
def _prop_kernel(xs_hbm, rows_hbm, cols_hbm, out_hbm,
                 ridx, cidx, buf0, buf1, acc, zbuf, sem0, sem1):
    c = lax.axis_index("c")
    s = lax.axis_index("s")
    wid = s * NC + c

    def zf(i, _):
        for k in range(HP // 16):
            zbuf[i, pl.ds(k * 16, 16)] = jnp.zeros((16,), jnp.float32)
        return 0

    lax.fori_loop(0, ZR, zf, 0)

    def zc(i, _):
        pltpu.sync_copy(zbuf, acc.at[pl.ds(s * RPT + i * ZR, ZR), :])
        return 0

    lax.fori_loop(0, RPT // ZR, zc, 0)
    pltpu.sync_copy(rows_hbm.at[wid], ridx)
    pltpu.sync_copy(cols_hbm.at[wid], cidx)
    plsc.subcore_barrier()

    pltpu.async_copy(xs_hbm.at[ridx.at[0]], buf0, sem0)
    pltpu.async_copy(xs_hbm.at[ridx.at[1]], buf1, sem1)

    def body(j2, _):
        j = 2 * j2
        pltpu.make_async_copy(xs_hbm.at[ridx.at[j]], buf0, sem0).wait()
        pltpu.sync_copy(buf0, acc.at[cidx.at[j]], add=True)
        pltpu.async_copy(xs_hbm.at[ridx.at[j + 2]], buf0, sem0)
        pltpu.make_async_copy(xs_hbm.at[ridx.at[j + 1]], buf1, sem1).wait()
        pltpu.sync_copy(buf1, acc.at[cidx.at[j + 1]], add=True)
        pltpu.async_copy(xs_hbm.at[ridx.at[j + 3]], buf1, sem1)
        return 0

    lax.fori_loop(0, NB // 2 - 1, body, 0)
    j = NB - 2
    pltpu.make_async_copy(xs_hbm.at[ridx.at[j]], buf0, sem0).wait()
    pltpu.sync_copy(buf0, acc.at[cidx.at[j]], add=True)
    pltpu.make_async_copy(xs_hbm.at[ridx.at[j + 1]], buf1, sem1).wait()
    pltpu.sync_copy(buf1, acc.at[cidx.at[j + 1]], add=True)
    plsc.subcore_barrier()
    pltpu.sync_copy(acc.at[pl.ds(s * RPT, RPT), :],
                    out_hbm.at[c, pl.ds(s * RPT, RPT), :])


def _mlp_body(xt_ref, xo_ref, deg_ref, w1, b1, w2, b2,
              ht_ref, xs1_ref, dinv_ref, qsum_ref):
    i = pl.program_id(0)

    def mlp(x):
        h = jnp.maximum(
            jnp.dot(x, w1[...], preferred_element_type=jnp.float32) + b1[...], 0.0)
        return jnp.maximum(
            jnp.dot(h, w2[...], preferred_element_type=jnp.float32) + b2[...], 0.0)

    ht = mlp(xt_ref[...])
    ho = mlp(xo_ref[...])
    deg = deg_ref[0][:, 0:1] + deg_ref[1][:, 0:1]
    dinv = lax.rsqrt(jnp.maximum(deg, 1.0))
    ht_ref[...] = ht
    xs1_ref[...] = ht * dinv
    dinv_ref[...] = dinv

    @pl.when(i == 0)
    def _():
        qsum_ref[...] = jnp.zeros((8, H), jnp.float32)

    qsum_ref[0:1, :] = qsum_ref[0:1, :] + jnp.sum(ho, axis=0, keepdims=True)


_mlp_call = pl.pallas_call(
    _mlp_body,
    grid=(GRID,),
    in_specs=[
        pl.BlockSpec((BR, IN), lambda i: (i, 0)),
        pl.BlockSpec((BR, IN), lambda i: (i, 0)),
        pl.BlockSpec((NC, BR, DW), lambda i: (0, i, 0)),
        pl.BlockSpec((IN, H), lambda i: (0, 0)),
        pl.BlockSpec((1, H), lambda i: (0, 0)),
        pl.BlockSpec((H, H), lambda i: (0, 0)),
        pl.BlockSpec((1, H), lambda i: (0, 0)),
    ],
    out_specs=[
        pl.BlockSpec((BR, H), lambda i: (i, 0)),
        pl.BlockSpec((BR, HP), lambda i: (i, 0)),
        pl.BlockSpec((BR, 1), lambda i: (i, 0)),
        pl.BlockSpec((8, H), lambda i: (0, 0)),
    ],
    out_shape=[
        jax.ShapeDtypeStruct((N, H), jnp.float32),
        jax.ShapeDtypeStruct((N, HP), jnp.float32),
        jax.ShapeDtypeStruct((N, 1), jnp.float32),
        jax.ShapeDtypeStruct((8, H), jnp.float32),
    ],
)


def _combine_body(p_ref, ht_ref, dinv_ref, af_ref, x1_ref, xs2_ref):
    dinv = dinv_ref[...]
    out = (p_ref[0] + p_ref[1]) * dinv
    ht = ht_ref[...]
    af_ref[...] = ht + out
    x1 = ht - out
    x1_ref[...] = x1
    xs2_ref[...] = x1 * dinv


_combine_call = pl.pallas_call(
    _combine_body,
    grid=(GRID,),
    in_specs=[
        pl.BlockSpec((NC, BR, HP), lambda i: (0, i, 0)),
        pl.BlockSpec((BR, H), lambda i: (i, 0)),
        pl.BlockSpec((BR, 1), lambda i: (i, 0)),
    ],
    out_specs=[
        pl.BlockSpec((BR, H), lambda i: (i, 0)),
        pl.BlockSpec((BR, H), lambda i: (i, 0)),
        pl.BlockSpec((BR, HP), lambda i: (i, 0)),
    ],
    out_shape=[
        jax.ShapeDtypeStruct((N, H), jnp.float32),
        jax.ShapeDtypeStruct((N, H), jnp.float32),
        jax.ShapeDtypeStruct((N, HP), jnp.float32),
    ],
)


def _attn_body(af_ref, x1_ref, p2_ref, dinv_ref, qsum_ref,
               wq, bq, wk, bk, wv, bv, wd, bd, z_ref, xhat_ref):
    dinv = dinv_ref[...]
    x2 = x1_ref[...] - (p2_ref[0] + p2_ref[1]) * dinv
    qg = jnp.dot(qsum_ref[0:1, :] * (1.0 / N), wq[...],
                 preferred_element_type=jnp.float32) + bq[...]
    scale = 1.0 / (H ** 0.5)

    def kv(l):
        kj = jnp.dot(l, wk[...], preferred_element_type=jnp.float32) + bk[...]
        sj = lax.dot_general(kj, qg, (((1,), (1,)), ((), ())),
                             preferred_element_type=jnp.float32)
        vj = jnp.dot(l, wv[...], preferred_element_type=jnp.float32) + bv[...]
        return scale * sj, vj

    s0, v0 = kv(af_ref[...])
    s1, v1 = kv(x1_ref[...])
    s2, v2 = kv(x2)
    m = jnp.maximum(jnp.maximum(s0, s1), s2)
    e0 = jnp.exp(s0 - m)
    e1 = jnp.exp(s1 - m)
    e2 = jnp.exp(s2 - m)
    z = (e0 * v0 + e1 * v1 + e2 * v2) / (e0 + e1 + e2)
    z_ref[...] = z
    xhat_ref[...] = jnp.dot(z, wd[...], preferred_element_type=jnp.float32) + bd[...]


_attn_call = pl.pallas_call(
    _attn_body,
    grid=(GRID,),
    in_specs=[
        pl.BlockSpec((BR, H), lambda i: (i, 0)),
        pl.BlockSpec((BR, H), lambda i: (i, 0)),
        pl.BlockSpec((NC, BR, HP), lambda i: (0, i, 0)),
        pl.BlockSpec((BR, 1), lambda i: (i, 0)),
        pl.BlockSpec((8, H), lambda i: (0, 0)),
        pl.BlockSpec((H, QK), lambda i: (0, 0)),
        pl.BlockSpec((1, QK), lambda i: (0, 0)),
        pl.BlockSpec((H, QK), lambda i: (0, 0)),
        pl.BlockSpec((1, QK), lambda i: (0, 0)),
        pl.BlockSpec((H, H), lambda i: (0, 0)),
        pl.BlockSpec((1, H), lambda i: (0, 0)),
        pl.BlockSpec((H, IN), lambda i: (0, 0)),
        pl.BlockSpec((1, IN), lambda i: (0, 0)),
    ],
    out_specs=[
        pl.BlockSpec((BR, H), lambda i: (i, 0)),
        pl.BlockSpec((BR, IN), lambda i: (i, 0)),
    ],
    out_shape=[
        jax.ShapeDtypeStruct((N, H), jnp.float32),
        jax.ShapeDtypeStruct((N, IN), jnp.float32),
    ],
)


def _adj_body(zi_ref, zfull_ref, out_ref):
    out_ref[...] = lax.dot_general(
        zi_ref[...], zfull_ref[...], (((1,), (1,)), ((), ())),
        preferred_element_type=jnp.float32)


_adj_call = pl.pallas_call(
    _adj_body,
    grid=(N // BRA,),
    in_specs=[
        pl.BlockSpec((BRA, H), lambda i: (i, 0)),
        pl.BlockSpec((N, H), lambda i: (0, 0)),
    ],
    out_specs=pl.BlockSpec((BRA, N), lambda i: (i, 0)),
    out_shape=jax.ShapeDtypeStruct((N, N), jnp.float32),
)


def kernel(x_train, edge_index, x_ori, mask,
           W1, b1, W2, b2, Wq, bq, Wk, bk, Wv, bv, Wd, bd):
    er = edge_index.reshape(2, NW, NB, BLK)
    rows = er[0]
    cols = er[1]

    degp = _deg_kernel(rows)

    h_train, xs1, dinv, qsum = _mlp_call(
        x_train, x_ori, degp, W1, b1.reshape(1, H), W2, b2.reshape(1, H))

    p1 = _prop_kernel(xs1, rows, cols)
    a_feat, x1, xs2 = _combine_call(p1, h_train, dinv)
    p2 = _prop_kernel(xs2, rows, cols)

    z, x_hat = _attn_call(
        a_feat, x1, p2, dinv, qsum,
        Wq, bq.reshape(1, QK), Wk, bk.reshape(1, QK),
        Wv, bv.reshape(1, H), Wd, bd.reshape(1, IN))

    adj_hat = _adj_call(z, z)
    return (x_hat, adj_hat)
